# baseline (speedup 1.0000x reference)
import functools

import jax
import jax.numpy as jnp
from jax import lax
from jax.experimental import pallas as pl
from jax.experimental.pallas import tpu as pltpu
from jax.experimental.pallas import tpu_sc as plsc


def _add_body(a_ref, b_ref, o_ref):
    o_ref[...] = a_ref[...] + b_ref[...]


def _dense_add(a, b, rows, mt=None):
    m, d = a.shape
    mt = m if mt is None else mt
    assert mt % rows == 0
    return pl.pallas_call(
        _add_body,
        grid=(mt // rows,),
        in_specs=[
            pl.BlockSpec((rows, d), lambda i: (i, 0)),
            pl.BlockSpec((rows, d), lambda i: (i, 0)),
        ],
        out_specs=pl.BlockSpec((rows, d), lambda i: (i, 0)),
        out_shape=jax.ShapeDtypeStruct((m, d), a.dtype),
        compiler_params=pltpu.CompilerParams(
            dimension_semantics=("parallel",),
        ),
    )(a, b)


@functools.cache
def _make_sc_tail_add(m, d, mt, chunk):
    num_cores, num_subcores, lanes = 2, 16, 16
    nw = num_cores * num_subcores
    ms = m - mt
    per_w = ms // nw
    nchunks = per_w // chunk
    assert per_w % chunk == 0 and ms % nw == 0
    mesh = plsc.VectorSubcoreMesh(
        core_axis_name="c", subcore_axis_name="s",
        num_cores=num_cores, num_subcores=num_subcores,
    )

    @functools.partial(
        pl.kernel,
        mesh=mesh,
        out_type=(),
        scratch_types=[
            pltpu.VMEM((chunk, d), jnp.float32),
            pltpu.VMEM((chunk, d), jnp.float32),
            pltpu.VMEM((chunk, d), jnp.float32),
            pltpu.VMEM((chunk, d), jnp.float32),
            pltpu.SemaphoreType.DMA,
            pltpu.SemaphoreType.DMA,
            pltpu.SemaphoreType.DMA,
            pltpu.SemaphoreType.DMA,
        ],
    )
    def sc_tail_add(x_hbm, c1_hbm, out_ref, xa0, ca0, xa1, ca1, si0, si1,
                    so0, so1):
        wid = lax.axis_index("s") * num_cores + lax.axis_index("c")
        base0 = wid * per_w + mt
        bufs = ((xa0, ca0, si0, so0), (xa1, ca1, si1, so1))

        def issue_in(c, b):
            xa, ca, si, _ = bufs[b]
            src = pl.ds(base0 + c * chunk, chunk)
            pltpu.async_copy(x_hbm.at[src], xa, si)
            pltpu.async_copy(c1_hbm.at[src], ca, si)

        def wait_out(b):
            xa, _, _, so = bufs[b]
            pltpu.make_async_copy(xa, out_ref.at[pl.ds(0, chunk)], so).wait()

        issue_in(0, 0)
        for c in range(nchunks):
            b = c & 1
            xa, ca, si, so = bufs[b]
            if c + 1 < nchunks:
                if c >= 1:
                    wait_out(b ^ 1)
                issue_in(c + 1, b ^ 1)
            rows = pl.ds(base0 + c * chunk, chunk)
            pltpu.make_async_copy(x_hbm.at[rows], xa, si).wait()
            pltpu.make_async_copy(c1_hbm.at[rows], ca, si).wait()

            unroll = 4

            @pl.loop(0, chunk // unroll)
            def _row(i):
                r = i * unroll
                for u in range(unroll):
                    for k in range(d // lanes):
                        sl = pl.ds(k * lanes, lanes)
                        xa[r + u, sl] = xa[r + u, sl] + ca[r + u, sl]

            pltpu.async_copy(xa, out_ref.at[rows], so)
        wait_out((nchunks - 1) & 1)
        if nchunks >= 2:
            wait_out(nchunks & 1)

    return sc_tail_add


@functools.cache
def _make_sc_scatter(b, d):
    num_cores, num_subcores, lanes = 2, 16, 16
    nw = num_cores * num_subcores
    b_per_w = b // nw
    mesh = plsc.VectorSubcoreMesh(
        core_axis_name="c", subcore_axis_name="s",
        num_cores=num_cores, num_subcores=num_subcores,
    )

    @functools.partial(
        pl.kernel,
        mesh=mesh,
        out_type=(),
        scratch_types=[
            pltpu.VMEM((b_per_w,), jnp.int32),
            pltpu.VMEM((b_per_w // 2, d), jnp.float32),
            pltpu.VMEM((b_per_w // 2, d), jnp.float32),
            pltpu.SemaphoreType.DMA,
        ],
    )
    def sc_scatter(y_hbm, c2_hbm, idx_hbm, out_ref, idx_v, y_v, c2_v, sem):
        wid = lax.axis_index("s") * num_cores + lax.axis_index("c")
        base = wid * b_per_w
        half = b_per_w // 2
        pltpu.sync_copy(idx_hbm.at[pl.ds(base, b_per_w)], idx_v)
        for h in range(2):
            r0 = base + h * half
            pltpu.sync_copy(y_hbm.at[pl.ds(r0, half)], y_v)
            pltpu.sync_copy(c2_hbm.at[pl.ds(r0, half)], c2_v)

            @pl.loop(0, half)
            def _row(i):
                for k in range(d // lanes):
                    sl = pl.ds(k * lanes, lanes)
                    y_v[i, sl] = y_v[i, sl] + c2_v[i, sl]

            @pl.loop(0, half // lanes)
            def _grp(g):
                vec = idx_v[pl.ds(h * half + g * lanes, lanes)]
                for k in range(lanes):
                    pltpu.async_copy(
                        y_v.at[pl.ds(g * lanes + k, 1)],
                        out_ref.at[pl.ds(vec[k], 1)],
                        sem,
                    )

            pltpu.make_async_copy(y_v, out_ref.at[pl.ds(0, half)], sem).wait()

    return sc_scatter


def kernel(x, y, c1, c2, index):
    m, d = x.shape
    mt = 250400
    dense = _dense_add(x, c1, rows=800, mt=mt)
    out_ref = jax.new_ref(dense)
    _make_sc_tail_add(m, d, mt, chunk=120)(x, c1, out_ref)
    _make_sc_scatter(y.shape[0], y.shape[1])(y, c2, index, out_ref)
    return out_ref[...]


# speedup vs baseline: 1.0492x; 1.0492x over previous
import functools

import jax
import jax.numpy as jnp
from jax import lax
from jax.experimental import pallas as pl
from jax.experimental.pallas import tpu as pltpu
from jax.experimental.pallas import tpu_sc as plsc


def _add_body(a_ref, b_ref, o_ref):
    o_ref[...] = a_ref[...] + b_ref[...]


def _dense_add(a, b, rows, mt=None):
    m, d = a.shape
    mt = m if mt is None else mt
    assert mt % rows == 0
    return pl.pallas_call(
        _add_body,
        grid=(mt // rows,),
        in_specs=[
            pl.BlockSpec((rows, d), lambda i: (i, 0)),
            pl.BlockSpec((rows, d), lambda i: (i, 0)),
        ],
        out_specs=pl.BlockSpec((rows, d), lambda i: (i, 0)),
        out_shape=jax.ShapeDtypeStruct((m, d), a.dtype),
        compiler_params=pltpu.CompilerParams(
            dimension_semantics=("parallel",),
        ),
    )(a, b)


@functools.cache
def _make_sc_tail_add(m, d, mt, chunk):
    num_cores, num_subcores, lanes = 2, 16, 16
    nw = num_cores * num_subcores
    ms = m - mt
    per_w = ms // nw
    nchunks = per_w // chunk
    assert per_w % chunk == 0 and ms % nw == 0
    mesh = plsc.VectorSubcoreMesh(
        core_axis_name="c", subcore_axis_name="s",
        num_cores=num_cores, num_subcores=num_subcores,
    )

    @functools.partial(
        pl.kernel,
        mesh=mesh,
        out_type=(),
        scratch_types=[
            pltpu.VMEM((chunk, d), jnp.float32),
            pltpu.VMEM((chunk, d), jnp.float32),
            pltpu.VMEM((chunk, d), jnp.float32),
            pltpu.VMEM((chunk, d), jnp.float32),
            pltpu.SemaphoreType.DMA,
            pltpu.SemaphoreType.DMA,
            pltpu.SemaphoreType.DMA,
            pltpu.SemaphoreType.DMA,
        ],
    )
    def sc_tail_add(x_hbm, c1_hbm, out_ref, xa0, ca0, xa1, ca1, si0, si1,
                    so0, so1):
        wid = lax.axis_index("s") * num_cores + lax.axis_index("c")
        base0 = wid * per_w + mt
        bufs = ((xa0, ca0, si0, so0), (xa1, ca1, si1, so1))

        def issue_in(c, b):
            xa, ca, si, _ = bufs[b]
            src = pl.ds(base0 + c * chunk, chunk)
            pltpu.async_copy(x_hbm.at[src], xa, si)
            pltpu.async_copy(c1_hbm.at[src], ca, si)

        def wait_out(b):
            xa, _, _, so = bufs[b]
            pltpu.make_async_copy(xa, out_ref.at[pl.ds(0, chunk)], so).wait()

        issue_in(0, 0)
        for c in range(nchunks):
            b = c & 1
            xa, ca, si, so = bufs[b]
            if c + 1 < nchunks:
                if c >= 1:
                    wait_out(b ^ 1)
                issue_in(c + 1, b ^ 1)
            rows = pl.ds(base0 + c * chunk, chunk)
            pltpu.make_async_copy(x_hbm.at[rows], xa, si).wait()
            pltpu.make_async_copy(c1_hbm.at[rows], ca, si).wait()

            unroll = 4

            @pl.loop(0, chunk // unroll)
            def _row(i):
                r = i * unroll
                for u in range(unroll):
                    for k in range(d // lanes):
                        sl = pl.ds(k * lanes, lanes)
                        xa[r + u, sl] = xa[r + u, sl] + ca[r + u, sl]

            pltpu.async_copy(xa, out_ref.at[rows], so)
        wait_out((nchunks - 1) & 1)
        if nchunks >= 2:
            wait_out(nchunks & 1)

    return sc_tail_add


@functools.cache
def _make_sc_scatter(b, d):
    num_cores, num_subcores, lanes = 2, 16, 16
    nw = num_cores * num_subcores
    b_per_w = b // nw
    mesh = plsc.VectorSubcoreMesh(
        core_axis_name="c", subcore_axis_name="s",
        num_cores=num_cores, num_subcores=num_subcores,
    )

    @functools.partial(
        pl.kernel,
        mesh=mesh,
        out_type=(),
        scratch_types=[
            pltpu.VMEM((b_per_w,), jnp.int32),
            pltpu.VMEM((b_per_w // 2, d), jnp.float32),
            pltpu.VMEM((b_per_w // 2, d), jnp.float32),
            pltpu.SemaphoreType.DMA,
        ],
    )
    def sc_scatter(y_hbm, c2_hbm, idx_hbm, out_ref, idx_v, y_v, c2_v, sem):
        wid = lax.axis_index("s") * num_cores + lax.axis_index("c")
        base = wid * b_per_w
        half = b_per_w // 2
        pltpu.sync_copy(idx_hbm.at[pl.ds(base, b_per_w)], idx_v)
        for h in range(2):
            r0 = base + h * half
            pltpu.sync_copy(y_hbm.at[pl.ds(r0, half)], y_v)
            pltpu.sync_copy(c2_hbm.at[pl.ds(r0, half)], c2_v)

            @pl.loop(0, half)
            def _row(i):
                for k in range(d // lanes):
                    sl = pl.ds(k * lanes, lanes)
                    y_v[i, sl] = y_v[i, sl] + c2_v[i, sl]

            @pl.loop(0, half // lanes)
            def _grp(g):
                vec = idx_v[pl.ds(h * half + g * lanes, lanes)]
                for k in range(lanes):
                    pltpu.async_copy(
                        y_v.at[pl.ds(g * lanes + k, 1)],
                        out_ref.at[pl.ds(vec[k], 1)],
                        sem,
                    )

            pltpu.make_async_copy(y_v, out_ref.at[pl.ds(0, half)], sem).wait()

    return sc_scatter


def kernel(x, y, c1, c2, index):
    m, d = x.shape
    mt = 252192
    dense = _dense_add(x, c1, rows=1184, mt=mt)
    out_ref = jax.new_ref(dense)
    _make_sc_tail_add(m, d, mt, chunk=176)(x, c1, out_ref)
    _make_sc_scatter(y.shape[0], y.shape[1])(y, c2, index, out_ref)
    return out_ref[...]


# speedup vs baseline: 1.0721x; 1.0218x over previous
import functools

import jax
import jax.numpy as jnp
from jax import lax
from jax.experimental import pallas as pl
from jax.experimental.pallas import tpu as pltpu
from jax.experimental.pallas import tpu_sc as plsc


def _add_body(a_ref, b_ref, o_ref):
    o_ref[...] = a_ref[...] + b_ref[...]


def _dense_add(a, b, rows, mt=None):
    m, d = a.shape
    mt = m if mt is None else mt
    assert mt % rows == 0
    return pl.pallas_call(
        _add_body,
        grid=(mt // rows,),
        in_specs=[
            pl.BlockSpec((rows, d), lambda i: (i, 0)),
            pl.BlockSpec((rows, d), lambda i: (i, 0)),
        ],
        out_specs=pl.BlockSpec((rows, d), lambda i: (i, 0)),
        out_shape=jax.ShapeDtypeStruct((m, d), a.dtype),
        compiler_params=pltpu.CompilerParams(
            dimension_semantics=("parallel",),
        ),
    )(a, b)


@functools.cache
def _make_sc_tail_add(m, d, mt, chunk):
    num_cores, num_subcores, lanes = 2, 16, 16
    nw = num_cores * num_subcores
    ms = m - mt
    per_w = ms // nw
    nchunks = per_w // chunk
    assert per_w % chunk == 0 and ms % nw == 0
    mesh = plsc.VectorSubcoreMesh(
        core_axis_name="c", subcore_axis_name="s",
        num_cores=num_cores, num_subcores=num_subcores,
    )

    @functools.partial(
        pl.kernel,
        mesh=mesh,
        out_type=(),
        scratch_types=[
            pltpu.VMEM((chunk, d), jnp.float32),
            pltpu.VMEM((chunk, d), jnp.float32),
            pltpu.VMEM((chunk, d), jnp.float32),
            pltpu.VMEM((chunk, d), jnp.float32),
            pltpu.SemaphoreType.DMA,
            pltpu.SemaphoreType.DMA,
            pltpu.SemaphoreType.DMA,
            pltpu.SemaphoreType.DMA,
        ],
    )
    def sc_tail_add(x_hbm, c1_hbm, out_ref, xa0, ca0, xa1, ca1, si0, si1,
                    so0, so1):
        wid = lax.axis_index("s") * num_cores + lax.axis_index("c")
        base0 = wid * per_w + mt
        bufs = ((xa0, ca0, si0, so0), (xa1, ca1, si1, so1))

        def issue_in(c, b):
            xa, ca, si, _ = bufs[b]
            src = pl.ds(base0 + c * chunk, chunk)
            pltpu.async_copy(x_hbm.at[src], xa, si)
            pltpu.async_copy(c1_hbm.at[src], ca, si)

        def wait_out(b):
            xa, _, _, so = bufs[b]
            pltpu.make_async_copy(xa, out_ref.at[pl.ds(0, chunk)], so).wait()

        issue_in(0, 0)
        for c in range(nchunks):
            b = c & 1
            xa, ca, si, so = bufs[b]
            if c + 1 < nchunks:
                if c >= 1:
                    wait_out(b ^ 1)
                issue_in(c + 1, b ^ 1)
            rows = pl.ds(base0 + c * chunk, chunk)
            pltpu.make_async_copy(x_hbm.at[rows], xa, si).wait()
            pltpu.make_async_copy(c1_hbm.at[rows], ca, si).wait()

            unroll = 4

            @pl.loop(0, chunk // unroll)
            def _row(i):
                r = i * unroll
                for u in range(unroll):
                    for k in range(d // lanes):
                        sl = pl.ds(k * lanes, lanes)
                        xa[r + u, sl] = xa[r + u, sl] + ca[r + u, sl]

            pltpu.async_copy(xa, out_ref.at[rows], so)
        wait_out((nchunks - 1) & 1)
        if nchunks >= 2:
            wait_out(nchunks & 1)

    return sc_tail_add


@functools.cache
def _make_sc_scatter(b, d):
    num_cores, num_subcores, lanes = 2, 16, 16
    nw = num_cores * num_subcores
    b_per_w = b // nw
    mesh = plsc.VectorSubcoreMesh(
        core_axis_name="c", subcore_axis_name="s",
        num_cores=num_cores, num_subcores=num_subcores,
    )

    @functools.partial(
        pl.kernel,
        mesh=mesh,
        out_type=(),
        scratch_types=[
            pltpu.VMEM((b_per_w,), jnp.int32),
            pltpu.VMEM((b_per_w // 2, d), jnp.float32),
            pltpu.VMEM((b_per_w // 2, d), jnp.float32),
            pltpu.SemaphoreType.DMA,
        ],
    )
    def sc_scatter(y_hbm, c2_hbm, idx_hbm, out_ref, idx_v, y_v, c2_v, sem):
        wid = lax.axis_index("s") * num_cores + lax.axis_index("c")
        base = wid * b_per_w
        half = b_per_w // 2
        pltpu.sync_copy(idx_hbm.at[pl.ds(base, b_per_w)], idx_v)
        for h in range(2):
            r0 = base + h * half
            pltpu.sync_copy(y_hbm.at[pl.ds(r0, half)], y_v)
            pltpu.sync_copy(c2_hbm.at[pl.ds(r0, half)], c2_v)

            @pl.loop(0, half)
            def _row(i):
                for k in range(d // lanes):
                    sl = pl.ds(k * lanes, lanes)
                    y_v[i, sl] = y_v[i, sl] + c2_v[i, sl]

            @pl.loop(0, half // lanes)
            def _grp(g):
                vec = idx_v[pl.ds(h * half + g * lanes, lanes)]
                for k in range(lanes):
                    pltpu.async_copy(
                        y_v.at[pl.ds(g * lanes + k, 1)],
                        out_ref.at[pl.ds(vec[k], 1)],
                        sem,
                    )

            pltpu.make_async_copy(y_v, out_ref.at[pl.ds(0, half)], sem).wait()

    return sc_scatter


def kernel(x, y, c1, c2, index):
    m, d = x.shape
    mt = 49440
    dense = _dense_add(x, c1, rows=480, mt=mt)
    out_ref = jax.new_ref(dense)
    _make_sc_tail_add(m, d, mt, chunk=176)(x, c1, out_ref)
    _make_sc_scatter(y.shape[0], y.shape[1])(y, c2, index, out_ref)
    return out_ref[...]
